# Initial kernel scaffold; baseline (speedup 1.0000x reference)
import functools

import jax
import jax.numpy as jnp
from jax import lax
from jax.experimental import pallas as pl
from jax.experimental.pallas import tpu as pltpu
from jax.experimental.pallas import tpu_sc as plsc

NC = 2
NS = 16
NW = NC * NS

V = 256
D = 64

B = 1024 * 200 * 4
BPW = B // NW
CH = 512
NCHUNK = BPW // CH

_mesh = plsc.VectorSubcoreMesh(core_axis_name="c", subcore_axis_name="s")


@functools.partial(
    pl.kernel,
    mesh=_mesh,
    out_type=jax.ShapeDtypeStruct((B * D,), jnp.float32),
    scratch_types=[
        pltpu.VMEM((V * D,), jnp.float32),
        pltpu.VMEM((BPW,), jnp.int32),
        pltpu.VMEM((CH * D,), jnp.float32),
        pltpu.VMEM((CH * D,), jnp.float32),
        pltpu.SemaphoreType.DMA,
        pltpu.SemaphoreType.DMA,
    ],
)
def _gather_kernel(idx_hbm, w_hbm, out_hbm, table_v, idx_v, rows0, rows1,
                   sem0, sem1):
    wid = lax.axis_index("s") * NC + lax.axis_index("c")
    base = wid * BPW

    pltpu.sync_copy(w_hbm, table_v)
    pltpu.sync_copy(idx_hbm.at[pl.ds(base, BPW)], idx_v)

    bufs = ((rows0, sem0), (rows1, sem1))

    def fill_and_send(c, rows_b, sem_b):
        out_sl = out_hbm.at[pl.ds((base + c * CH) * D, CH * D)]

        @pl.when(c >= 2)
        def _():
            pltpu.make_async_copy(rows_b, out_sl, sem_b).wait()

        def row_fn(r, _):
            off = idx_v[c * CH + r] * D
            rb = r * D
            for k in range(D // 16):
                rows_b[pl.ds(rb + k * 16, 16)] = table_v[pl.ds(off + k * 16, 16)]
            return 0

        lax.fori_loop(0, CH, row_fn, 0, unroll=2)
        pltpu.async_copy(rows_b, out_sl, sem_b)

    def round_fn(i, _):
        c0 = i * 2
        for b in range(2):
            fill_and_send(c0 + b, *bufs[b])
        return 0

    lax.fori_loop(0, NCHUNK // 2, round_fn, 0)

    for b in range(2):
        c_last = NCHUNK - 2 + b
        out_sl = out_hbm.at[pl.ds((base + c_last * CH) * D, CH * D)]
        pltpu.make_async_copy(bufs[b][0], out_sl, bufs[b][1]).wait()


def kernel(byte_indices, W):
    batch, seq, four = byte_indices.shape
    idx_flat = byte_indices.reshape(-1).astype(jnp.int32)
    w_flat = W.reshape(-1).astype(jnp.float32)
    out_flat = _gather_kernel(idx_flat, w_flat)
    return out_flat.reshape(batch, seq, four * W.shape[1])


# speedup vs baseline: 4.3487x; 4.3487x over previous
import functools

import jax
import jax.numpy as jnp
from jax import lax
from jax.experimental import pallas as pl
from jax.experimental.pallas import tpu as pltpu
from jax.experimental.pallas import tpu_sc as plsc

NC = 2
NS = 16
NW = NC * NS

V = 256
D = 64

B = 1024 * 200 * 4
BPW = B // NW
CH = 512
NCHUNK = BPW // CH

_mesh = plsc.VectorSubcoreMesh(core_axis_name="c", subcore_axis_name="s")


@functools.partial(
    pl.kernel,
    mesh=_mesh,
    out_type=jax.ShapeDtypeStruct((B * D,), jnp.float32),
    scratch_types=[
        pltpu.VMEM((V * D,), jnp.float32),
        pltpu.VMEM((BPW,), jnp.int32),
        pltpu.VMEM((CH * D,), jnp.float32),
        pltpu.VMEM((CH * D,), jnp.float32),
        pltpu.SemaphoreType.DMA,
        pltpu.SemaphoreType.DMA,
    ],
)
def _gather_kernel(idx_hbm, w_hbm, out_hbm, table_v, idx_v, rows0, rows1,
                   sem0, sem1):
    wid = lax.axis_index("s") * NC + lax.axis_index("c")
    base = wid * BPW

    pltpu.sync_copy(w_hbm, table_v)
    pltpu.sync_copy(idx_hbm.at[pl.ds(base, BPW)], idx_v)

    bufs = ((rows0, sem0), (rows1, sem1))

    def fill_and_send(c, rows_b, sem_b):
        out_sl = out_hbm.at[pl.ds((base + c * CH) * D, CH * D)]

        @pl.when(c >= 2)
        def _():
            pltpu.make_async_copy(rows_b, out_sl, sem_b).wait()

        def row_fn(g, _):
            iv = idx_v[pl.ds(c * CH + g * 16, 16)] * D
            rb = g * (16 * D)
            for j in range(16):
                off = iv[j]
                for k in range(D // 16):
                    rows_b[pl.ds(rb + j * D + k * 16, 16)] = (
                        table_v[pl.ds(off + k * 16, 16)])
            return 0

        lax.fori_loop(0, CH // 16, row_fn, 0)
        pltpu.async_copy(rows_b, out_sl, sem_b)

    def round_fn(i, _):
        c0 = i * 2
        for b in range(2):
            fill_and_send(c0 + b, *bufs[b])
        return 0

    lax.fori_loop(0, NCHUNK // 2, round_fn, 0)

    for b in range(2):
        c_last = NCHUNK - 2 + b
        out_sl = out_hbm.at[pl.ds((base + c_last * CH) * D, CH * D)]
        pltpu.make_async_copy(bufs[b][0], out_sl, bufs[b][1]).wait()


def kernel(byte_indices, W):
    batch, seq, four = byte_indices.shape
    idx_flat = byte_indices.reshape(-1).astype(jnp.int32)
    w_flat = W.reshape(-1).astype(jnp.float32)
    out_flat = _gather_kernel(idx_flat, w_flat)
    return out_flat.reshape(batch, seq, four * W.shape[1])
